# Initial kernel scaffold; baseline (speedup 1.0000x reference)
import jax
import jax.numpy as jnp
from jax.experimental import pallas as pl


def kernel(x, edge_index, edge_attr, batch_ids, W_sl0, b_sl0, W_sl, b_sl, W_gat, a_src, a_dst, W_edge, a_edge, b_gat, W_xc, b_xc, W_cc, b_cc, W_at, b_at, W_out, b_out):
    raise NotImplementedError("write your pallas kernel here")


# speedup vs baseline: 1.0063x; 1.0063x over previous
import jax
import jax.numpy as jnp
from jax.experimental import pallas as pl

_N_TYPES = 5
_N_LAYERS = 3
_NUM_GRAPHS = 256


def _gat_edge(x_src, x_dst, ei, ea, W, asrc, adst, We, ae, b):
    n_dst = x_dst.shape[0]
    hs = x_src @ W
    hd = x_dst @ W
    src = ei[0]
    dst = ei[1]
    alpha = (hs * asrc).sum(-1)[src] + (hd * adst).sum(-1)[dst] + ((ea @ We) * ae).sum(-1)
    alpha = jax.nn.leaky_relu(alpha, negative_slope=0.2)
    amax = jax.ops.segment_max(alpha, dst, num_segments=n_dst)
    amax = jnp.where(jnp.isfinite(amax), amax, 0.0)
    ex = jnp.exp(alpha - amax[dst])
    denom = jax.ops.segment_sum(ex, dst, num_segments=n_dst)
    a = ex / (denom[dst] + 1e-16)
    out = jax.ops.segment_sum(a[:, None] * hs[src], dst, num_segments=n_dst)
    return out + b


def _final_pool_kernel(xs_ref, bb_ref, wout_ref, bout_ref, out_ref):
    n = xs_ref.shape[1]
    ssum = jnp.zeros((_NUM_GRAPHS, 128), jnp.float32)
    cnt = jnp.zeros((_NUM_GRAPHS,), jnp.float32)
    for t in range(_N_TYPES):
        bb = bb_ref[t, :]
        iota = jax.lax.broadcasted_iota(jnp.int32, (n, _NUM_GRAPHS), 1)
        oh = (iota == bb[:, None]).astype(jnp.float32)
        ssum = ssum + jax.lax.dot_general(
            oh, xs_ref[t], (((0,), (0,)), ((), ())),
            preferred_element_type=jnp.float32)
        cnt = cnt + oh.sum(0)
    pooled = ssum / jnp.maximum(cnt, 1.0)[:, None]
    out_ref[...] = jax.nn.sigmoid(pooled @ wout_ref[...] + bout_ref[0])


def _final_pool(xs, batch_ids, W_out, b_out):
    return pl.pallas_call(
        _final_pool_kernel,
        out_shape=jax.ShapeDtypeStruct((_NUM_GRAPHS, 1), jnp.float32),
    )(xs, batch_ids, W_out, b_out)


def kernel(x, edge_index, edge_attr, batch_ids, W_sl0, b_sl0, W_sl, b_sl, W_gat, a_src, a_dst, W_edge, a_edge, b_gat, W_xc, b_xc, W_cc, b_cc, W_at, b_at, W_out, b_out):
    xs = [x[t] for t in range(_N_TYPES)]
    for L in range(_N_LAYERS):
        if L == 0:
            xs = [jax.nn.leaky_relu(xs[t] @ W_sl0[t] + b_sl0[t]) for t in range(_N_TYPES)]
        else:
            xs = [jax.nn.leaky_relu(xs[t] @ W_sl[L - 1, t] + b_sl[L - 1, t]) for t in range(_N_TYPES)]
        comms = []
        for dt in range(_N_TYPES):
            outs = []
            for st in range(_N_TYPES):
                e = st * _N_TYPES + dt
                outs.append(_gat_edge(xs[st], xs[dt], edge_index[e], edge_attr[e], W_gat[L, e], a_src[L, e], a_dst[L, e], W_edge[L, e], a_edge[L, e], b_gat[L, e]))
            comms.append(jnp.max(jnp.stack(outs, 0), axis=0))
        new_xs = []
        for t in range(_N_TYPES):
            c = jax.nn.leaky_relu(comms[t])
            xt = jnp.concatenate([xs[t] @ W_xc[L, t] + b_xc[L, t], c @ W_cc[L, t] + b_cc[L, t]], axis=-1)
            att = xt @ W_at[L, t] + b_at[L, t]
            new_xs.append(xt + jax.nn.sigmoid(att) * xt)
        xs = new_xs
    xs = [jax.nn.leaky_relu(xs[t] @ W_sl[2, t] + b_sl[2, t]) for t in range(_N_TYPES)]
    xs = [jax.nn.leaky_relu(xs[t] @ W_sl[3, t] + b_sl[3, t]) for t in range(_N_TYPES)]
    return _final_pool(jnp.stack(xs, 0), batch_ids, W_out, b_out)


# speedup vs baseline: 4.5211x; 4.4927x over previous
import functools

import jax
import jax.numpy as jnp
from jax import lax
from jax.experimental import pallas as pl
from jax.experimental.pallas import tpu as pltpu
from jax.experimental.pallas import tpu_sc as plsc

_NT = 5
_NG = 256
_N = 2000
_E = 6400
_H = 128

_NSUB = 16
_EPP = 512
_EP = _EPP * _NSUB
_NP = 2048
_RPS = _NP // _NSUB
_EPC = 13


def _lrelu(x, slope):
    return jnp.where(x >= 0, x, x * slope)


def _stage_a_body(x_ref, W_ref, b_ref, Wgs_ref, asrc_ref, Wgd_ref, adst_ref,
                  eaT_ref, We_ref, ae_ref,
                  xn_ref, S_ref, aeg_ref):
    xn = _lrelu(jnp.dot(x_ref[0], W_ref[0], preferred_element_type=jnp.float32)
                + b_ref[0], 0.01)
    xn_ref[0] = xn
    cols = []
    for dt in range(_NT):
        row = asrc_ref[0, dt][None, :]
        cols.append(jnp.sum(Wgs_ref[0, dt] * row, axis=1, keepdims=True))
        erow = ae_ref[0, dt][None, :]
        we = jnp.sum(We_ref[0, dt] * erow, axis=1, keepdims=True)
        aeg_ref[0, dt] = jnp.sum(eaT_ref[0, dt] * we, axis=0, keepdims=True)
    for st in range(_NT):
        drow = adst_ref[st, 0]
        cols.append(jnp.sum(Wgd_ref[st, 0] * drow, axis=1, keepdims=True))
    cols.append(jnp.zeros((_H, _H - 2 * _NT), jnp.float32))
    M = jnp.concatenate(cols, axis=1)
    S_ref[0] = jnp.dot(xn, M, preferred_element_type=jnp.float32)


def _stage_a(x, W, b, Wg_r, asrc_r, adst_r, eaT_r, We_r, ae_r):
    din = x.shape[-1]
    f32 = jnp.float32
    return pl.pallas_call(
        _stage_a_body,
        grid=(_NT,),
        in_specs=[
            pl.BlockSpec((1, _N, din), lambda t: (t, 0, 0)),
            pl.BlockSpec((1, din, _H), lambda t: (t, 0, 0)),
            pl.BlockSpec((1, 1, _H), lambda t: (t, 0, 0)),
            pl.BlockSpec((1, _NT, _H, _H), lambda t: (t, 0, 0, 0)),
            pl.BlockSpec((1, _NT, _H), lambda t: (t, 0, 0)),
            pl.BlockSpec((_NT, 1, _H, _H), lambda t: (0, t, 0, 0)),
            pl.BlockSpec((_NT, 1, 1, _H), lambda t: (0, t, 0, 0)),
            pl.BlockSpec((1, _NT, 4, _E), lambda t: (t, 0, 0, 0)),
            pl.BlockSpec((1, _NT, 4, _H), lambda t: (t, 0, 0, 0)),
            pl.BlockSpec((1, _NT, _H), lambda t: (t, 0, 0)),
        ],
        out_specs=[
            pl.BlockSpec((1, _N, _H), lambda t: (t, 0, 0)),
            pl.BlockSpec((1, _N, _H), lambda t: (t, 0, 0)),
            pl.BlockSpec((1, _NT, 1, _E), lambda t: (t, 0, 0, 0)),
        ],
        out_shape=[
            jax.ShapeDtypeStruct((_NT, _N, _H), f32),
            jax.ShapeDtypeStruct((_NT, _N, _H), f32),
            jax.ShapeDtypeStruct((_NT, _NT, 1, _E), f32),
        ],
    )(x, W, b.reshape(_NT, 1, _H), Wg_r, asrc_r, Wg_r,
      adst_r.reshape(_NT, _NT, 1, _H), eaT_r, We_r, ae_r)


def _sc_edge_body(xflat, ssrc, sdst, aeg, esrc, edst, agg_out,
                  s_src_l, s_dst_l, aeg_l, src_l, dst_l, src2d, dst2d,
                  ex_l, a_l, den_l, rows, zbuf, z816, iota_r, den_sh,
                  agg_sh, sem):
    c = lax.axis_index("c")
    s = lax.axis_index("s")
    base = s * _EPP
    zf = jnp.zeros((16,), jnp.float32)

    def _zb(i, carry):
        r = i // 8
        k = i % 8
        zbuf[r, pl.ds(k * 16, 16)] = zf
        return carry
    lax.fori_loop(0, _RPS * 8, _zb, 0)
    for r in range(8):
        z816[r, :] = zf
    for k in range(8):
        iota_r[0, pl.ds(k * 16, 16)] = lax.iota(jnp.int32, 16) + k * 16

    def _per_edge_type(i, carry):
        e = jnp.minimum(c * _EPC + i, 24)
        st = e // _NT

        pltpu.sync_copy(ssrc.at[e], s_src_l)
        pltpu.sync_copy(sdst.at[e], s_dst_l)
        pltpu.sync_copy(aeg.at[e].at[pl.ds(base, _EPP)], aeg_l)
        pltpu.sync_copy(esrc.at[e].at[pl.ds(base, _EPP)], src_l)
        pltpu.sync_copy(edst.at[e].at[pl.ds(base, _EPP)], dst_l)

        def _zd(r, carry2):
            den_l[r] = zf
            return carry2
        lax.fori_loop(0, 128, _zd, 0)
        pltpu.sync_copy(zbuf, agg_sh.at[pl.ds(s * _RPS, _RPS)])
        pltpu.sync_copy(z816, den_sh.at[pl.ds(s * 8, 8)])
        plsc.subcore_barrier()

        def _p1(g, carry2):
            o = g * 16
            vs = src_l[pl.ds(o, 16)]
            vd = dst_l[pl.ds(o, 16)]
            a1 = plsc.load_gather(s_src_l, [vs])
            a2 = plsc.load_gather(s_dst_l, [jnp.minimum(vd, _N - 1)])
            al = a1 + a2 + aeg_l[pl.ds(o, 16)]
            al = jnp.where(al >= 0, al, al * 0.2)
            ex = jnp.exp(al)
            ex_l[pl.ds(o, 16)] = ex
            plsc.addupdate_scatter(den_l, [vd // 16, vd % 16], ex)
            return carry2
        lax.fori_loop(0, _EPP // 16, _p1, 0)

        pltpu.sync_copy(den_l, den_sh.at[iota_r.at[0]], add=True)
        plsc.subcore_barrier()
        pltpu.sync_copy(den_sh, den_l)

        def _p2(g, carry2):
            o = g * 16
            vd = dst_l[pl.ds(o, 16)]
            dv = plsc.load_gather(den_l, [vd // 16, vd % 16])
            a_l[pl.ds(o, 16)] = ex_l[pl.ds(o, 16)] / (dv + 1e-16)
            return carry2
        lax.fori_loop(0, _EPP // 16, _p2, 0)

        def _idx(g, carry2):
            o = g * 16
            src2d[g // 8, pl.ds((g % 8) * 16, 16)] = src_l[pl.ds(o, 16)] + st * _N
            dst2d[g // 8, pl.ds((g % 8) * 16, 16)] = dst_l[pl.ds(o, 16)]
            return carry2
        lax.fori_loop(0, _EPP // 16, _idx, 0)

        descs = [pltpu.async_copy(xflat.at[src2d.at[j]],
                                  rows.at[pl.ds(j * 128, 128)], sem)
                 for j in range(_EPP // 128)]
        for d in descs:
            d.wait()

        def _scale(i2, carry2):
            ab = plsc.load_gather(a_l, [jnp.full((16,), 0, jnp.int32) + i2])
            for k in range(8):
                rows[i2, pl.ds(k * 16, 16)] = rows[i2, pl.ds(k * 16, 16)] * ab
            return carry2
        lax.fori_loop(0, _EPP, _scale, 0)

        for j in range(_EPP // 128):
            pltpu.sync_copy(rows.at[pl.ds(j * 128, 128)],
                            agg_sh.at[dst2d.at[j]], add=True)
        plsc.subcore_barrier()

        pltpu.sync_copy(agg_sh.at[pl.ds(s * _RPS, _RPS)], rows.at[pl.ds(0, _RPS)])
        pltpu.sync_copy(rows.at[pl.ds(0, _RPS)],
                        agg_out.at[e].at[pl.ds(s * _RPS, _RPS)])
        return carry

    lax.fori_loop(0, _EPC, _per_edge_type, 0)


@functools.cache
def _make_sc_edge_aggregate():
    @functools.partial(
        pl.kernel,
        mesh=plsc.VectorSubcoreMesh(core_axis_name="c", subcore_axis_name="s"),
        out_type=jax.ShapeDtypeStruct((_NT * _NT, _NP, _H), jnp.float32),
        compiler_params=pltpu.CompilerParams(needs_layout_passes=False),
        scratch_types=[
            pltpu.VMEM((_N,), jnp.float32),
            pltpu.VMEM((_N,), jnp.float32),
            pltpu.VMEM((_EPP,), jnp.float32),
            pltpu.VMEM((_EPP,), jnp.int32),
            pltpu.VMEM((_EPP,), jnp.int32),
            pltpu.VMEM((4, 128), jnp.int32),
            pltpu.VMEM((4, 128), jnp.int32),
            pltpu.VMEM((_EPP,), jnp.float32),
            pltpu.VMEM((_EPP,), jnp.float32),
            pltpu.VMEM((128, 16), jnp.float32),
            pltpu.VMEM((_EPP, _H), jnp.float32),
            pltpu.VMEM((_RPS, _H), jnp.float32),
            pltpu.VMEM((8, 16), jnp.float32),
            pltpu.VMEM((1, 128), jnp.int32),
            pltpu.VMEM_SHARED((128, 16), jnp.float32),
            pltpu.VMEM_SHARED((_NP, _H), jnp.float32),
            pltpu.SemaphoreType.DMA,
        ],
    )
    def _sc_edge_aggregate(xflat, ssrc, sdst, aeg, esrc, edst, agg_out, *rest):
        _sc_edge_body(xflat, ssrc, sdst, aeg, esrc, edst, agg_out, *rest)

    return _sc_edge_aggregate


def _stage_c_body(agg_ref, Wg_ref, bg_ref, x_ref, Wxc_ref, bxc_ref,
                  Wcc_ref, bcc_ref, Wat_ref, bat_ref, out_ref):
    comms = None
    for st in range(_NT):
        o = jnp.dot(agg_ref[st, 0], Wg_ref[st, 0],
                    preferred_element_type=jnp.float32) + bg_ref[st, 0]
        comms = o if comms is None else jnp.maximum(comms, o)
    cc = _lrelu(comms, 0.01)
    xv = x_ref[0]
    left = jnp.dot(xv, Wxc_ref[0], preferred_element_type=jnp.float32) + bxc_ref[0]
    right = jnp.dot(cc, Wcc_ref[0], preferred_element_type=jnp.float32) + bcc_ref[0]
    xt = jnp.concatenate([left, right], axis=1)
    att = jnp.dot(xt, Wat_ref[0], preferred_element_type=jnp.float32) + bat_ref[0]
    out_ref[0] = xt + jax.nn.sigmoid(att) * xt


def _stage_c(agg_r, Wg_r, bg_r, xn, Wxc, bxc, Wcc, bcc, Wat, bat):
    f32 = jnp.float32
    return pl.pallas_call(
        _stage_c_body,
        grid=(_NT,),
        in_specs=[
            pl.BlockSpec((_NT, 1, _N, _H), lambda t: (0, t, 0, 0)),
            pl.BlockSpec((_NT, 1, _H, _H), lambda t: (0, t, 0, 0)),
            pl.BlockSpec((_NT, 1, 1, _H), lambda t: (0, t, 0, 0)),
            pl.BlockSpec((1, _N, _H), lambda t: (t, 0, 0)),
            pl.BlockSpec((1, _H, _H // 2), lambda t: (t, 0, 0)),
            pl.BlockSpec((1, 1, _H // 2), lambda t: (t, 0, 0)),
            pl.BlockSpec((1, _H, _H // 2), lambda t: (t, 0, 0)),
            pl.BlockSpec((1, 1, _H // 2), lambda t: (t, 0, 0)),
            pl.BlockSpec((1, _H, _H), lambda t: (t, 0, 0)),
            pl.BlockSpec((1, 1, _H), lambda t: (t, 0, 0)),
        ],
        out_specs=pl.BlockSpec((1, _N, _H), lambda t: (t, 0, 0)),
        out_shape=jax.ShapeDtypeStruct((_NT, _N, _H), f32),
    )(agg_r, Wg_r, bg_r.reshape(_NT, _NT, 1, _H), xn,
      Wxc, bxc.reshape(_NT, 1, _H // 2), Wcc, bcc.reshape(_NT, 1, _H // 2),
      Wat, bat.reshape(_NT, 1, _H))


def _stage_d_body(x_ref, W2_ref, b2_ref, W3_ref, b3_ref, bb_ref,
                  wout_ref, bout_ref, out_ref):
    b2 = b2_ref[...]
    b3 = b3_ref[...]
    ssum = jnp.zeros((_NG, _H), jnp.float32)
    cnt = jnp.zeros((_NG,), jnp.float32)
    for t in range(_NT):
        y = _lrelu(jnp.dot(x_ref[t], W2_ref[t], preferred_element_type=jnp.float32)
                   + b2[t:t + 1], 0.01)
        y = _lrelu(jnp.dot(y, W3_ref[t], preferred_element_type=jnp.float32)
                   + b3[t:t + 1], 0.01)
        bb = bb_ref[t, :]
        iota = lax.broadcasted_iota(jnp.int32, (_N, _NG), 1)
        oh = (iota == bb[:, None]).astype(jnp.float32)
        ssum = ssum + lax.dot_general(oh, y, (((0,), (0,)), ((), ())),
                                      preferred_element_type=jnp.float32)
        cnt = cnt + oh.sum(0)
    pooled = ssum / jnp.maximum(cnt, 1.0)[:, None]
    out_ref[...] = jax.nn.sigmoid(
        jnp.dot(pooled, wout_ref[...], preferred_element_type=jnp.float32)
        + bout_ref[0])


def _stage_d(xs, W2, b2, W3, b3, batch_ids, W_out, b_out):
    return pl.pallas_call(
        _stage_d_body,
        out_shape=jax.ShapeDtypeStruct((_NG, 1), jnp.float32),
    )(xs, W2, b2, W3, b3, batch_ids, W_out, b_out)


def kernel(x, edge_index, edge_attr, batch_ids, W_sl0, b_sl0, W_sl, b_sl,
           W_gat, a_src, a_dst, W_edge, a_edge, b_gat, W_xc, b_xc, W_cc,
           b_cc, W_at, b_at, W_out, b_out):
    def _pad_edges(arr, cval):
        a3 = arr.reshape(_NT * _NT, _NSUB, _E // _NSUB)
        a3 = jnp.pad(a3, ((0, 0), (0, 0), (0, _EPP - _E // _NSUB)),
                     constant_values=cval)
        return a3.reshape(_NT * _NT, _EP)

    esrc = _pad_edges(edge_index[:, 0, :], 0)
    edst = _pad_edges(edge_index[:, 1, :], _N)
    eaT_r = edge_attr.transpose(0, 2, 1).reshape(_NT, _NT, 4, _E)

    xs = x
    for L in range(3):
        Wg_r = W_gat[L].reshape(_NT, _NT, _H, _H)
        asrc_r = a_src[L].reshape(_NT, _NT, _H)
        adst_r = a_dst[L].reshape(_NT, _NT, _H)
        We_r = W_edge[L].reshape(_NT, _NT, 4, _H)
        ae_r = a_edge[L].reshape(_NT, _NT, _H)
        if L == 0:
            Wl, bl = W_sl0, b_sl0
        else:
            Wl, bl = W_sl[L - 1], b_sl[L - 1]
        xn, S, aeg4 = _stage_a(xs, Wl, bl, Wg_r, asrc_r, adst_r,
                               eaT_r, We_r, ae_r)
        ssrc = S[:, :, :_NT].transpose(0, 2, 1).reshape(_NT * _NT, _N)
        sdst = S[:, :, _NT:2 * _NT].transpose(2, 0, 1).reshape(_NT * _NT, _N)
        aeg_p = _pad_edges(aeg4.reshape(_NT * _NT, _E), 0.0)
        agg = _make_sc_edge_aggregate()(
            xn.reshape(_NT * _N, _H), ssrc, sdst, aeg_p, esrc, edst)
        xs = _stage_c(agg[:, :_N, :].reshape(_NT, _NT, _N, _H), Wg_r,
                      b_gat[L].reshape(_NT, _NT, _H), xn,
                      W_xc[L], b_xc[L], W_cc[L], b_cc[L], W_at[L], b_at[L])
    return _stage_d(xs, W_sl[2], b_sl[2], W_sl[3], b_sl[3],
                    batch_ids, W_out, b_out)


# speedup vs baseline: 4.9077x; 1.0855x over previous
import functools

import jax
import jax.numpy as jnp
from jax import lax
from jax.experimental import pallas as pl
from jax.experimental.pallas import tpu as pltpu
from jax.experimental.pallas import tpu_sc as plsc

_NT = 5
_NG = 256
_N = 2000
_E = 6400
_H = 128

_NSUB = 16
_EPP = 512
_EP = _EPP * _NSUB
_NP = 2048
_RPS = _NP // _NSUB
_EPC = 13


def _lrelu(x, slope):
    return jnp.where(x >= 0, x, x * slope)


def _stage_a_body(x_ref, W_ref, b_ref, Wgs_ref, asrc_ref, Wgd_ref, adst_ref,
                  eaT_ref, We_ref, ae_ref,
                  xn_ref, S_ref, aeg_ref):
    xn = _lrelu(jnp.dot(x_ref[0], W_ref[0], preferred_element_type=jnp.float32)
                + b_ref[0], 0.01)
    xn_ref[0] = xn
    cols = []
    for dt in range(_NT):
        row = asrc_ref[0, dt][None, :]
        cols.append(jnp.sum(Wgs_ref[0, dt] * row, axis=1, keepdims=True))
        erow = ae_ref[0, dt][None, :]
        we = jnp.sum(We_ref[0, dt] * erow, axis=1, keepdims=True)
        aeg_ref[0, dt] = jnp.sum(eaT_ref[0, dt] * we, axis=0, keepdims=True)
    for st in range(_NT):
        drow = adst_ref[st, 0]
        cols.append(jnp.sum(Wgd_ref[st, 0] * drow, axis=1, keepdims=True))
    cols.append(jnp.zeros((_H, _H - 2 * _NT), jnp.float32))
    M = jnp.concatenate(cols, axis=1)
    S_ref[0] = jnp.dot(xn, M, preferred_element_type=jnp.float32)


def _stage_a(x, W, b, Wg_r, asrc_r, adst_r, eaT_r, We_r, ae_r):
    din = x.shape[-1]
    f32 = jnp.float32
    return pl.pallas_call(
        _stage_a_body,
        grid=(_NT,),
        in_specs=[
            pl.BlockSpec((1, _N, din), lambda t: (t, 0, 0)),
            pl.BlockSpec((1, din, _H), lambda t: (t, 0, 0)),
            pl.BlockSpec((1, 1, _H), lambda t: (t, 0, 0)),
            pl.BlockSpec((1, _NT, _H, _H), lambda t: (t, 0, 0, 0)),
            pl.BlockSpec((1, _NT, _H), lambda t: (t, 0, 0)),
            pl.BlockSpec((_NT, 1, _H, _H), lambda t: (0, t, 0, 0)),
            pl.BlockSpec((_NT, 1, 1, _H), lambda t: (0, t, 0, 0)),
            pl.BlockSpec((1, _NT, 4, _E), lambda t: (t, 0, 0, 0)),
            pl.BlockSpec((1, _NT, 4, _H), lambda t: (t, 0, 0, 0)),
            pl.BlockSpec((1, _NT, _H), lambda t: (t, 0, 0)),
        ],
        out_specs=[
            pl.BlockSpec((1, _N, _H), lambda t: (t, 0, 0)),
            pl.BlockSpec((1, _N, _H), lambda t: (t, 0, 0)),
            pl.BlockSpec((1, _NT, 1, _E), lambda t: (t, 0, 0, 0)),
        ],
        out_shape=[
            jax.ShapeDtypeStruct((_NT, _N, _H), f32),
            jax.ShapeDtypeStruct((_NT, _N, _H), f32),
            jax.ShapeDtypeStruct((_NT, _NT, 1, _E), f32),
        ],
    )(x, W, b.reshape(_NT, 1, _H), Wg_r, asrc_r, Wg_r,
      adst_r.reshape(_NT, _NT, 1, _H), eaT_r, We_r, ae_r)


def _sc_edge_body(xflat, ssrc, sdst, aeg, esrc, edst, agg_out,
                  s_src_l, s_dst_l, aeg_l, src_l, dst_l, src2d, dst2d,
                  ex_l, a_l, den_l, rows, zbuf, z816, iota_r, den_sh,
                  agg_sh, sem_in, sem_z, sem_g, sem_s):
    c = lax.axis_index("c")
    s = lax.axis_index("s")
    base = s * _EPP
    zf = jnp.zeros((16,), jnp.float32)

    def _zb(i, carry):
        r = i // 8
        k = i % 8
        zbuf[r, pl.ds(k * 16, 16)] = zf
        return carry
    lax.fori_loop(0, _RPS * 8, _zb, 0)
    for r in range(8):
        z816[r, :] = zf
    for k in range(8):
        iota_r[0, pl.ds(k * 16, 16)] = lax.iota(jnp.int32, 16) + k * 16

    def _per_edge_type(i, carry):
        e = jnp.minimum(c * _EPC + i, 24)
        st = e // _NT

        d_in = [
            pltpu.async_copy(ssrc.at[e], s_src_l, sem_in),
            pltpu.async_copy(sdst.at[e], s_dst_l, sem_in),
            pltpu.async_copy(aeg.at[e].at[pl.ds(base, _EPP)], aeg_l, sem_in),
            pltpu.async_copy(esrc.at[e].at[pl.ds(base, _EPP)], src_l, sem_in),
            pltpu.async_copy(edst.at[e].at[pl.ds(base, _EPP)], dst_l, sem_in),
        ]
        d_z = [
            pltpu.async_copy(zbuf, agg_sh.at[pl.ds(s * _RPS, _RPS)], sem_z),
            pltpu.async_copy(z816, den_sh.at[pl.ds(s * 8, 8)], sem_z),
        ]
        def _zd(r, carry2):
            den_l[r] = zf
            return carry2
        lax.fori_loop(0, 128, _zd, 0)
        for d in d_in:
            d.wait()

        def _idx(g, carry2):
            o = g * 16
            src2d[g // 8, pl.ds((g % 8) * 16, 16)] = src_l[pl.ds(o, 16)] + st * _N
            dst2d[g // 8, pl.ds((g % 8) * 16, 16)] = dst_l[pl.ds(o, 16)]
            return carry2
        lax.fori_loop(0, _EPP // 16, _idx, 0)
        d_g = [pltpu.async_copy(xflat.at[src2d.at[j]],
                                rows.at[pl.ds(j * 128, 128)], sem_g)
               for j in range(_EPP // 128)]

        def _p1(g, carry2):
            o = g * 16
            vs = src_l[pl.ds(o, 16)]
            vd = dst_l[pl.ds(o, 16)]
            a1 = plsc.load_gather(s_src_l, [vs])
            a2 = plsc.load_gather(s_dst_l, [jnp.minimum(vd, _N - 1)])
            al = a1 + a2 + aeg_l[pl.ds(o, 16)]
            al = jnp.where(al >= 0, al, al * 0.2)
            ex = jnp.exp(al)
            ex_l[pl.ds(o, 16)] = ex
            plsc.addupdate_scatter(den_l, [vd // 16, vd % 16], ex)
            return carry2
        lax.fori_loop(0, _EPP // 16, _p1, 0)

        for d in d_z:
            d.wait()
        plsc.subcore_barrier()
        pltpu.sync_copy(den_l, den_sh.at[iota_r.at[0]], add=True)
        plsc.subcore_barrier()
        pltpu.sync_copy(den_sh, den_l)

        def _p2(g, carry2):
            o = g * 16
            vd = dst_l[pl.ds(o, 16)]
            dv = plsc.load_gather(den_l, [vd // 16, vd % 16])
            a_l[pl.ds(o, 16)] = ex_l[pl.ds(o, 16)] / (dv + 1e-16)
            return carry2
        lax.fori_loop(0, _EPP // 16, _p2, 0)

        for d in d_g:
            d.wait()

        def _scale(i2, carry2):
            ab = plsc.load_gather(a_l, [jnp.full((16,), 0, jnp.int32) + i2])
            for k in range(8):
                rows[i2, pl.ds(k * 16, 16)] = rows[i2, pl.ds(k * 16, 16)] * ab
            return carry2
        lax.fori_loop(0, _EPP, _scale, 0)

        d_s = [pltpu.async_copy(rows.at[pl.ds(j * 128, 128)],
                                agg_sh.at[dst2d.at[j]], sem_s, add=True)
               for j in range(_EPP // 128)]
        for d in d_s:
            d.wait()
        plsc.subcore_barrier()

        pltpu.sync_copy(agg_sh.at[pl.ds(s * _RPS, _RPS)],
                        agg_out.at[e].at[pl.ds(s * _RPS, _RPS)])
        return carry

    lax.fori_loop(0, _EPC, _per_edge_type, 0)


@functools.cache
def _make_sc_edge_aggregate():
    @functools.partial(
        pl.kernel,
        mesh=plsc.VectorSubcoreMesh(core_axis_name="c", subcore_axis_name="s"),
        out_type=jax.ShapeDtypeStruct((_NT * _NT, _NP, _H), jnp.float32),
        compiler_params=pltpu.CompilerParams(needs_layout_passes=False),
        scratch_types=[
            pltpu.VMEM((_N,), jnp.float32),
            pltpu.VMEM((_N,), jnp.float32),
            pltpu.VMEM((_EPP,), jnp.float32),
            pltpu.VMEM((_EPP,), jnp.int32),
            pltpu.VMEM((_EPP,), jnp.int32),
            pltpu.VMEM((4, 128), jnp.int32),
            pltpu.VMEM((4, 128), jnp.int32),
            pltpu.VMEM((_EPP,), jnp.float32),
            pltpu.VMEM((_EPP,), jnp.float32),
            pltpu.VMEM((128, 16), jnp.float32),
            pltpu.VMEM((_EPP, _H), jnp.float32),
            pltpu.VMEM((_RPS, _H), jnp.float32),
            pltpu.VMEM((8, 16), jnp.float32),
            pltpu.VMEM((1, 128), jnp.int32),
            pltpu.VMEM_SHARED((128, 16), jnp.float32),
            pltpu.VMEM_SHARED((_NP, _H), jnp.float32),
            pltpu.SemaphoreType.DMA,
            pltpu.SemaphoreType.DMA,
            pltpu.SemaphoreType.DMA,
            pltpu.SemaphoreType.DMA,
        ],
    )
    def _sc_edge_aggregate(xflat, ssrc, sdst, aeg, esrc, edst, agg_out, *rest):
        _sc_edge_body(xflat, ssrc, sdst, aeg, esrc, edst, agg_out, *rest)

    return _sc_edge_aggregate


def _stage_c_body(agg_ref, Wg_ref, bg_ref, x_ref, Wxc_ref, bxc_ref,
                  Wcc_ref, bcc_ref, Wat_ref, bat_ref, out_ref):
    comms = None
    for st in range(_NT):
        o = jnp.dot(agg_ref[st, 0], Wg_ref[st, 0],
                    preferred_element_type=jnp.float32) + bg_ref[st, 0]
        comms = o if comms is None else jnp.maximum(comms, o)
    cc = _lrelu(comms, 0.01)
    xv = x_ref[0]
    left = jnp.dot(xv, Wxc_ref[0], preferred_element_type=jnp.float32) + bxc_ref[0]
    right = jnp.dot(cc, Wcc_ref[0], preferred_element_type=jnp.float32) + bcc_ref[0]
    xt = jnp.concatenate([left, right], axis=1)
    att = jnp.dot(xt, Wat_ref[0], preferred_element_type=jnp.float32) + bat_ref[0]
    out_ref[0] = xt + jax.nn.sigmoid(att) * xt


def _stage_c(agg_r, Wg_r, bg_r, xn, Wxc, bxc, Wcc, bcc, Wat, bat):
    f32 = jnp.float32
    return pl.pallas_call(
        _stage_c_body,
        grid=(_NT,),
        in_specs=[
            pl.BlockSpec((_NT, 1, _N, _H), lambda t: (0, t, 0, 0)),
            pl.BlockSpec((_NT, 1, _H, _H), lambda t: (0, t, 0, 0)),
            pl.BlockSpec((_NT, 1, 1, _H), lambda t: (0, t, 0, 0)),
            pl.BlockSpec((1, _N, _H), lambda t: (t, 0, 0)),
            pl.BlockSpec((1, _H, _H // 2), lambda t: (t, 0, 0)),
            pl.BlockSpec((1, 1, _H // 2), lambda t: (t, 0, 0)),
            pl.BlockSpec((1, _H, _H // 2), lambda t: (t, 0, 0)),
            pl.BlockSpec((1, 1, _H // 2), lambda t: (t, 0, 0)),
            pl.BlockSpec((1, _H, _H), lambda t: (t, 0, 0)),
            pl.BlockSpec((1, 1, _H), lambda t: (t, 0, 0)),
        ],
        out_specs=pl.BlockSpec((1, _N, _H), lambda t: (t, 0, 0)),
        out_shape=jax.ShapeDtypeStruct((_NT, _N, _H), f32),
    )(agg_r, Wg_r, bg_r.reshape(_NT, _NT, 1, _H), xn,
      Wxc, bxc.reshape(_NT, 1, _H // 2), Wcc, bcc.reshape(_NT, 1, _H // 2),
      Wat, bat.reshape(_NT, 1, _H))


def _stage_d_body(x_ref, W2_ref, b2_ref, W3_ref, b3_ref, bb_ref,
                  wout_ref, bout_ref, out_ref):
    b2 = b2_ref[...]
    b3 = b3_ref[...]
    ssum = jnp.zeros((_NG, _H), jnp.float32)
    cnt = jnp.zeros((_NG,), jnp.float32)
    for t in range(_NT):
        y = _lrelu(jnp.dot(x_ref[t], W2_ref[t], preferred_element_type=jnp.float32)
                   + b2[t:t + 1], 0.01)
        y = _lrelu(jnp.dot(y, W3_ref[t], preferred_element_type=jnp.float32)
                   + b3[t:t + 1], 0.01)
        bb = bb_ref[t, :]
        iota = lax.broadcasted_iota(jnp.int32, (_N, _NG), 1)
        oh = (iota == bb[:, None]).astype(jnp.float32)
        ssum = ssum + lax.dot_general(oh, y, (((0,), (0,)), ((), ())),
                                      preferred_element_type=jnp.float32)
        cnt = cnt + oh.sum(0)
    pooled = ssum / jnp.maximum(cnt, 1.0)[:, None]
    out_ref[...] = jax.nn.sigmoid(
        jnp.dot(pooled, wout_ref[...], preferred_element_type=jnp.float32)
        + bout_ref[0])


def _stage_d(xs, W2, b2, W3, b3, batch_ids, W_out, b_out):
    return pl.pallas_call(
        _stage_d_body,
        out_shape=jax.ShapeDtypeStruct((_NG, 1), jnp.float32),
    )(xs, W2, b2, W3, b3, batch_ids, W_out, b_out)


def kernel(x, edge_index, edge_attr, batch_ids, W_sl0, b_sl0, W_sl, b_sl,
           W_gat, a_src, a_dst, W_edge, a_edge, b_gat, W_xc, b_xc, W_cc,
           b_cc, W_at, b_at, W_out, b_out):
    def _pad_edges(arr, cval):
        a3 = arr.reshape(_NT * _NT, _NSUB, _E // _NSUB)
        a3 = jnp.pad(a3, ((0, 0), (0, 0), (0, _EPP - _E // _NSUB)),
                     constant_values=cval)
        return a3.reshape(_NT * _NT, _EP)

    esrc = _pad_edges(edge_index[:, 0, :], 0)
    edst = _pad_edges(edge_index[:, 1, :], _N)
    eaT_r = edge_attr.transpose(0, 2, 1).reshape(_NT, _NT, 4, _E)

    xs = x
    for L in range(3):
        Wg_r = W_gat[L].reshape(_NT, _NT, _H, _H)
        asrc_r = a_src[L].reshape(_NT, _NT, _H)
        adst_r = a_dst[L].reshape(_NT, _NT, _H)
        We_r = W_edge[L].reshape(_NT, _NT, 4, _H)
        ae_r = a_edge[L].reshape(_NT, _NT, _H)
        if L == 0:
            Wl, bl = W_sl0, b_sl0
        else:
            Wl, bl = W_sl[L - 1], b_sl[L - 1]
        xn, S, aeg4 = _stage_a(xs, Wl, bl, Wg_r, asrc_r, adst_r,
                               eaT_r, We_r, ae_r)
        ssrc = S[:, :, :_NT].transpose(0, 2, 1).reshape(_NT * _NT, _N)
        sdst = S[:, :, _NT:2 * _NT].transpose(2, 0, 1).reshape(_NT * _NT, _N)
        aeg_p = _pad_edges(aeg4.reshape(_NT * _NT, _E), 0.0)
        agg = _make_sc_edge_aggregate()(
            xn.reshape(_NT * _N, _H), ssrc, sdst, aeg_p, esrc, edst)
        xs = _stage_c(agg[:, :_N, :].reshape(_NT, _NT, _N, _H), Wg_r,
                      b_gat[L].reshape(_NT, _NT, _H), xn,
                      W_xc[L], b_xc[L], W_cc[L], b_cc[L], W_at[L], b_at[L])
    return _stage_d(xs, W_sl[2], b_sl[2], W_sl[3], b_sl[3],
                    batch_ids, W_out, b_out)
